# baseline (speedup 1.0000x reference)
import functools

import jax
import jax.numpy as jnp
from jax import lax
from jax.experimental import pallas as pl
from jax.experimental.pallas import tpu as pltpu
from jax.experimental.pallas import tpu_sc as plsc

_ROWS = 8192
_COLS = 1024


def _make_sc_copy():
    mesh = plsc.VectorSubcoreMesh(core_axis_name="c", subcore_axis_name="s")
    nw = mesh.num_cores * mesh.num_subcores
    rows_per_worker = _ROWS // nw

    chunk = 32
    n_chunks = rows_per_worker // chunk

    @functools.partial(
        pl.kernel,
        out_type=jax.ShapeDtypeStruct((_ROWS, _COLS), jnp.float32),
        mesh=mesh,
        scratch_types=[
            pltpu.VMEM((2, chunk, _COLS), jnp.float32),
            pltpu.SemaphoreType.DMA,
            pltpu.SemaphoreType.DMA,
            pltpu.SemaphoreType.DMA,
            pltpu.SemaphoreType.DMA,
        ],
    )
    def sc_copy(x_hbm, out_hbm, buf, ls0, ls1, ss0, ss1):
        load_sems = (ls0, ls1)
        store_sems = (ss0, ss1)
        wid = lax.axis_index("s") * mesh.num_cores + lax.axis_index("c")
        base = wid * rows_per_worker

        def load(i):
            return pltpu.async_copy(
                x_hbm.at[pl.ds(base + i * chunk, chunk)], buf.at[i % 2],
                load_sems[i % 2])

        def store(i):
            return pltpu.async_copy(
                buf.at[i % 2], out_hbm.at[pl.ds(base + i * chunk, chunk)],
                store_sems[i % 2])

        loads = {0: load(0), 1: load(1)}
        stores = {}
        for i in range(n_chunks):
            loads.pop(i).wait()
            stores[i] = store(i)
            j = i + 2
            if j < n_chunks:
                stores.pop(i).wait()
                loads[j] = load(j)
        for i in sorted(stores):
            stores[i].wait()

    return sc_copy


_CHUNK = 512
_DEPTH = 16
_NCHUNKS = _ROWS // _CHUNK


def _stream_body(x_ref, o_ref, buf, load_sems, store_sems):
    def load(i):
        return pltpu.make_async_copy(
            x_ref.at[pl.ds(i * _CHUNK, _CHUNK)], buf.at[i % _DEPTH],
            load_sems.at[i % _DEPTH])

    def store(i):
        return pltpu.make_async_copy(
            buf.at[i % _DEPTH], o_ref.at[pl.ds(i * _CHUNK, _CHUNK)],
            store_sems.at[i % _DEPTH])

    stores = {}
    for i in range(_DEPTH):
        load(i).start()
    for i in range(_NCHUNKS):
        load(i).wait()
        stores[i] = store(i)
        stores[i].start()
        j = i - 1
        k = j + _DEPTH
        if j >= 0 and k < _NCHUNKS:
            stores.pop(j).wait()
            load(k).start()
    for i in sorted(stores):
        stores[i].wait()


def kernel(x, embed_weight):
    del embed_weight
    flat = x.reshape(_ROWS, _COLS)
    out = pl.pallas_call(
        _stream_body,
        out_shape=jax.ShapeDtypeStruct(flat.shape, flat.dtype),
        in_specs=[pl.BlockSpec(memory_space=pl.ANY)],
        out_specs=pl.BlockSpec(memory_space=pl.ANY),
        scratch_shapes=[
            pltpu.VMEM((_DEPTH, _CHUNK, _COLS), jnp.float32),
            pltpu.SemaphoreType.DMA((_DEPTH,)),
            pltpu.SemaphoreType.DMA((_DEPTH,)),
        ],
    )(flat)
    return out.reshape(x.shape)


# speedup vs baseline: 1.0369x; 1.0369x over previous
import functools

import jax
import jax.numpy as jnp
from jax import lax
from jax.experimental import pallas as pl
from jax.experimental.pallas import tpu as pltpu
from jax.experimental.pallas import tpu_sc as plsc

_ROWS = 8192
_COLS = 1024


def _make_sc_copy():
    mesh = plsc.VectorSubcoreMesh(core_axis_name="c", subcore_axis_name="s")
    nw = mesh.num_cores * mesh.num_subcores
    rows_per_worker = _ROWS // nw

    chunk = 32
    n_chunks = rows_per_worker // chunk

    @functools.partial(
        pl.kernel,
        out_type=jax.ShapeDtypeStruct((_ROWS, _COLS), jnp.float32),
        mesh=mesh,
        scratch_types=[
            pltpu.VMEM((2, chunk, _COLS), jnp.float32),
            pltpu.SemaphoreType.DMA,
            pltpu.SemaphoreType.DMA,
            pltpu.SemaphoreType.DMA,
            pltpu.SemaphoreType.DMA,
        ],
    )
    def sc_copy(x_hbm, out_hbm, buf, ls0, ls1, ss0, ss1):
        load_sems = (ls0, ls1)
        store_sems = (ss0, ss1)
        wid = lax.axis_index("s") * mesh.num_cores + lax.axis_index("c")
        base = wid * rows_per_worker

        def load(i):
            return pltpu.async_copy(
                x_hbm.at[pl.ds(base + i * chunk, chunk)], buf.at[i % 2],
                load_sems[i % 2])

        def store(i):
            return pltpu.async_copy(
                buf.at[i % 2], out_hbm.at[pl.ds(base + i * chunk, chunk)],
                store_sems[i % 2])

        loads = {0: load(0), 1: load(1)}
        stores = {}
        for i in range(n_chunks):
            loads.pop(i).wait()
            stores[i] = store(i)
            j = i + 2
            if j < n_chunks:
                stores.pop(i).wait()
                loads[j] = load(j)
        for i in sorted(stores):
            stores[i].wait()

    return sc_copy


def _copy_body(x_ref, o_ref):
    o_ref[...] = x_ref[...]


def kernel(x, embed_weight):
    del embed_weight
    flat = x.reshape(_ROWS, _COLS)
    out = pl.pallas_call(
        _copy_body,
        out_shape=jax.ShapeDtypeStruct(flat.shape, flat.dtype),
        grid=(4,),
        in_specs=[pl.BlockSpec((2048, _COLS), lambda i: (i, 0))],
        out_specs=pl.BlockSpec((2048, _COLS), lambda i: (i, 0)),
        compiler_params=pltpu.CompilerParams(
            dimension_semantics=("arbitrary",),
        ),
    )(flat)
    return out.reshape(x.shape)
